# Initial kernel scaffold; baseline (speedup 1.0000x reference)
import functools

import jax
import jax.numpy as jnp
from jax import lax
from jax.experimental import pallas as pl
from jax.experimental.pallas import tpu as pltpu
from jax.experimental.pallas import tpu_sc as plsc

N_LEVELS = 8
LS = 12500
DEG = 64
E_PER = LS * DEG
P = 12512
NW = 32
CH = 400
TAIL_N = P - (NW - 1) * CH
TAIL_REAL = LS - (NW - 1) * CH
TAIL_E = TAIL_REAL * DEG

_MESH = dict(core_axis_name="c", subcore_axis_name="s")


def _wid():
    return lax.axis_index("s") * 2 + lax.axis_index("c")


def _make_level(l):
    eoff = l * E_PER
    sub = l * LS

    @functools.partial(
        pl.kernel,
        out_type=jax.ShapeDtypeStruct((P,), jnp.float32),
        mesh=plsc.VectorSubcoreMesh(**_MESH),
        scratch_types=[
            pltpu.VMEM((P,), jnp.float32),
            pltpu.VMEM((CH * DEG,), jnp.int32),
            pltpu.VMEM((CH * DEG,), jnp.float32),
            pltpu.VMEM((CH,), jnp.float32),
        ],
    )
    def level_fn(val_hbm, src_hbm, w_hbm, out_hbm, val_v, src_v, w_v, out_v):
        wid = _wid()
        base = wid * CH
        ebase = eoff + base * DEG
        pltpu.sync_copy(val_hbm, val_v)
        iota = lax.iota(jnp.int32, 16)

        def accum(posb, clamp):
            def kbody(k, acc):
                pos = posb + k
                s = plsc.load_gather(src_v, [pos])
                w = plsc.load_gather(w_v, [pos])
                vi = s - sub
                if clamp:
                    vi = jnp.minimum(jnp.maximum(vi, 0), P - 1)
                v = plsc.load_gather(val_v, [vi])
                return acc + v * w
            return lax.fori_loop(0, DEG, kbody, jnp.zeros((16,), jnp.float32),
                                 unroll=4)

        def do_group(g, clamp):
            acc = accum((g * 16 + iota) * DEG, clamp)
            out_v[pl.ds(g * 16, 16)] = jnp.where(acc >= 0, acc, 0.01 * acc)

        @pl.when(wid < NW - 1)
        def _():
            pltpu.sync_copy(src_hbm.at[pl.ds(ebase, CH * DEG)], src_v)
            pltpu.sync_copy(w_hbm.at[pl.ds(ebase, CH * DEG)], w_v)
            lax.fori_loop(0, CH // 16,
                          lambda g, _: (do_group(g, False), 0)[1], 0)
            pltpu.sync_copy(out_v, out_hbm.at[pl.ds(base, CH)])

        @pl.when(wid == NW - 1)
        def _():
            pltpu.sync_copy(src_hbm.at[pl.ds(ebase, TAIL_E)],
                            src_v.at[pl.ds(0, TAIL_E)])
            pltpu.sync_copy(w_hbm.at[pl.ds(ebase, TAIL_E)],
                            w_v.at[pl.ds(0, TAIL_E)])
            nfull = TAIL_REAL // 16
            lax.fori_loop(0, nfull,
                          lambda g, _: (do_group(g, False), 0)[1], 0)
            for g in range(nfull, TAIL_N // 16):
                do_group(jnp.int32(g), True)
            pltpu.sync_copy(out_v.at[pl.ds(0, TAIL_N)],
                            out_hbm.at[pl.ds(base, TAIL_N)])

    return level_fn


@functools.partial(
    pl.kernel,
    out_type=jax.ShapeDtypeStruct((P,), jnp.float32),
    mesh=plsc.VectorSubcoreMesh(**_MESH),
    scratch_types=[
        pltpu.VMEM((P,), jnp.float32),
        pltpu.VMEM((CH,), jnp.float32),
    ],
)
def _softmax_fn(val_hbm, out_hbm, val_v, out_v):
    wid = _wid()
    base = wid * CH
    pltpu.sync_copy(val_hbm, val_v)
    iota = lax.iota(jnp.int32, 16)
    mask = iota < (LS % 16)
    NEG = jnp.float32(-3.0e38)
    nfull = LS // 16

    last = val_v[pl.ds(nfull * 16, 16)]
    m = lax.fori_loop(
        0, nfull,
        lambda g, m: jnp.maximum(m, val_v[pl.ds(g * 16, 16)]),
        jnp.where(mask, last, NEG), unroll=4)
    mx = jnp.max(m)

    def ebody(g, sacc):
        e = jnp.exp(val_v[pl.ds(g * 16, 16)] - mx)
        val_v[pl.ds(g * 16, 16)] = e
        return sacc + e
    e_last = jnp.where(mask, jnp.exp(last - mx), 0.0)
    val_v[pl.ds(nfull * 16, 16)] = e_last
    s = lax.fori_loop(0, nfull, ebody, e_last, unroll=4)
    rinv = 1.0 / jnp.sum(s)

    def norm(ngroups):
        def obody(g, _):
            out_v[pl.ds(g * 16, 16)] = val_v[pl.ds(base + g * 16, 16)] * rinv
            return 0
        lax.fori_loop(0, ngroups, obody, 0)

    @pl.when(wid < NW - 1)
    def _():
        norm(CH // 16)
        pltpu.sync_copy(out_v, out_hbm.at[pl.ds(base, CH)])

    @pl.when(wid == NW - 1)
    def _():
        norm(TAIL_N // 16)
        pltpu.sync_copy(out_v.at[pl.ds(0, TAIL_N)],
                        out_hbm.at[pl.ds(base, TAIL_N)])


_LEVELS = [_make_level(l) for l in range(N_LEVELS - 1)]


@jax.jit
def _run(_input, edge_index, edge_weight):
    src_row = edge_index[0]
    val = jnp.concatenate([_input, jnp.zeros((P - LS,), jnp.float32)])
    for l in range(N_LEVELS - 1):
        val = _LEVELS[l](val, src_row, edge_weight)
    return _softmax_fn(val)[:LS]


def kernel(_input, edge_index, edge_weight):
    return _run(_input, edge_index, edge_weight)


# speedup vs baseline: 96.7993x; 96.7993x over previous
import functools

import jax
import jax.numpy as jnp
from jax import lax
from jax.experimental import pallas as pl
from jax.experimental.pallas import tpu as pltpu
from jax.experimental.pallas import tpu_sc as plsc

N_LEVELS = 8
LS = 12500
DEG = 64
E_PER = LS * DEG
P = 12512
NW = 32
CH = 400
TAIL_N = P - (NW - 1) * CH
TAIL_REAL = LS - (NW - 1) * CH
TAIL_E = TAIL_REAL * DEG

_MESH = dict(core_axis_name="c", subcore_axis_name="s")
_CPARAMS = pltpu.CompilerParams(needs_layout_passes=False)


def _wid():
    return lax.axis_index("s") * 2 + lax.axis_index("c")


def _make_level(l):
    eoff = l * E_PER
    sub = l * LS

    @functools.partial(
        pl.kernel,
        out_type=jax.ShapeDtypeStruct((P,), jnp.float32),
        mesh=plsc.VectorSubcoreMesh(**_MESH),
        compiler_params=_CPARAMS,
        scratch_types=[
            pltpu.VMEM((P,), jnp.float32),
            pltpu.VMEM((CH * DEG,), jnp.int32),
            pltpu.VMEM((CH * DEG,), jnp.float32),
            pltpu.VMEM((CH,), jnp.float32),
        ],
    )
    def level_fn(val_hbm, src_hbm, w_hbm, out_hbm, val_v, src_v, w_v, out_v):
        wid = _wid()
        base = wid * CH
        ebase = eoff + base * DEG
        pltpu.sync_copy(val_hbm, val_v)
        iota = lax.iota(jnp.int32, 16)

        def accum(posb, clamp):
            def kbody(k, acc):
                pos = posb + k
                s = plsc.load_gather(src_v, [pos])
                w = plsc.load_gather(w_v, [pos])
                vi = s - sub
                if clamp:
                    vi = jnp.minimum(jnp.maximum(vi, 0), P - 1)
                v = plsc.load_gather(val_v, [vi])
                return acc + v * w
            return lax.fori_loop(0, DEG, kbody, jnp.zeros((16,), jnp.float32),
                                 unroll=4)

        def do_group(g, clamp):
            acc = accum((g * 16 + iota) * DEG, clamp)
            out_v[pl.ds(g * 16, 16)] = jnp.where(acc >= 0, acc, 0.01 * acc)

        @pl.when(wid < NW - 1)
        def _():
            pltpu.sync_copy(src_hbm.at[pl.ds(ebase, CH * DEG)], src_v)
            pltpu.sync_copy(w_hbm.at[pl.ds(ebase, CH * DEG)], w_v)
            lax.fori_loop(0, CH // 16,
                          lambda g, _: (do_group(g, False), 0)[1], 0)
            pltpu.sync_copy(out_v, out_hbm.at[pl.ds(base, CH)])

        @pl.when(wid == NW - 1)
        def _():
            pltpu.sync_copy(src_hbm.at[pl.ds(ebase, TAIL_E)],
                            src_v.at[pl.ds(0, TAIL_E)])
            pltpu.sync_copy(w_hbm.at[pl.ds(ebase, TAIL_E)],
                            w_v.at[pl.ds(0, TAIL_E)])
            nfull = TAIL_REAL // 16
            lax.fori_loop(0, nfull,
                          lambda g, _: (do_group(g, False), 0)[1], 0)
            for g in range(nfull, TAIL_N // 16):
                do_group(jnp.int32(g), True)
            pltpu.sync_copy(out_v.at[pl.ds(0, TAIL_N)],
                            out_hbm.at[pl.ds(base, TAIL_N)])

    return level_fn


@functools.partial(
    pl.kernel,
    out_type=jax.ShapeDtypeStruct((P,), jnp.float32),
    mesh=plsc.VectorSubcoreMesh(**_MESH),
    compiler_params=_CPARAMS,
    scratch_types=[
        pltpu.VMEM((P,), jnp.float32),
        pltpu.VMEM((CH,), jnp.float32),
    ],
)
def _softmax_fn(val_hbm, out_hbm, val_v, out_v):
    wid = _wid()
    base = wid * CH
    pltpu.sync_copy(val_hbm, val_v)
    iota = lax.iota(jnp.int32, 16)
    mask = iota < (LS % 16)
    NEG = jnp.float32(-3.0e38)
    nfull = LS // 16

    last = val_v[pl.ds(nfull * 16, 16)]
    m = lax.fori_loop(
        0, nfull,
        lambda g, m: jnp.maximum(m, val_v[pl.ds(g * 16, 16)]),
        jnp.where(mask, last, NEG), unroll=4)
    mx = jnp.max(m)

    def ebody(g, sacc):
        e = jnp.exp(val_v[pl.ds(g * 16, 16)] - mx)
        val_v[pl.ds(g * 16, 16)] = e
        return sacc + e
    e_last = jnp.where(mask, jnp.exp(last - mx), 0.0)
    val_v[pl.ds(nfull * 16, 16)] = e_last
    s = lax.fori_loop(0, nfull, ebody, e_last, unroll=4)
    rinv = jnp.ones((16,), jnp.float32) / jnp.broadcast_to(jnp.sum(s), (16,))

    def norm(ngroups):
        def obody(g, _):
            out_v[pl.ds(g * 16, 16)] = val_v[pl.ds(base + g * 16, 16)] * rinv
            return 0
        lax.fori_loop(0, ngroups, obody, 0)

    @pl.when(wid < NW - 1)
    def _():
        norm(CH // 16)
        pltpu.sync_copy(out_v, out_hbm.at[pl.ds(base, CH)])

    @pl.when(wid == NW - 1)
    def _():
        norm(TAIL_N // 16)
        pltpu.sync_copy(out_v.at[pl.ds(0, TAIL_N)],
                        out_hbm.at[pl.ds(base, TAIL_N)])


_LEVELS = [_make_level(l) for l in range(N_LEVELS - 1)]


@jax.jit
def _run(_input, edge_index, edge_weight):
    src_row = edge_index[0]
    val = jnp.concatenate([_input, jnp.zeros((P - LS,), jnp.float32)])
    for l in range(N_LEVELS - 1):
        val = _LEVELS[l](val, src_row, edge_weight)
    return _softmax_fn(val)[:LS]


def kernel(_input, edge_index, edge_weight):
    return _run(_input, edge_index, edge_weight)
